# baseline (speedup 1.0000x reference)
import functools

import jax
import jax.numpy as jnp
from jax import lax
from jax.experimental import pallas as pl
from jax.experimental.pallas import tpu as pltpu
from jax.experimental.pallas import tpu_sc as plsc

E = 8
H = 1024
FFN = 2048
F2 = 2 * FFN
T = 2048
K = 2
NPAIR = K * T

TM = 128
NT = 40
P = NT * TM

NC = 2
NS = 16
NW = NC * NS
HALF = P // NC
SLOTS_W = HALF // NS
GCH = 80
CPW = NPAIR // NW
CCH = 64


def _plan_body(x_ref, wr_ref, inv_ref, w01_ref, st_ref, sc_ref):
    x = x_ref[...]
    wr = wr_ref[...]
    mix = jnp.dot(x, wr, preferred_element_type=jnp.float32)
    q, k, v = mix[:, 0:E], mix[:, E:2 * E], mix[:, 2 * E:3 * E]
    cols = []
    for i in range(E):
        a = q[:, i:i + 1] * k
        m = jnp.max(a, axis=1, keepdims=True)
        ex = jnp.exp(a - m)
        cols.append(jnp.sum(ex * v, axis=1, keepdims=True)
                    / jnp.sum(ex, axis=1, keepdims=True))
    logits = jnp.concatenate(cols, axis=1)
    iota8 = lax.broadcasted_iota(jnp.int32, (T, E), 1)
    l0 = jnp.max(logits, axis=1, keepdims=True)
    i0 = jnp.min(jnp.where(logits == l0, iota8, E), axis=1, keepdims=True)
    rest = jnp.where(iota8 == i0, -jnp.inf, logits)
    l1 = jnp.max(rest, axis=1, keepdims=True)
    i1 = jnp.min(jnp.where(rest == l1, iota8, E), axis=1, keepdims=True)
    w0 = 1.0 / (1.0 + jnp.exp(l1 - l0))

    oh0 = (iota8 == i0).astype(jnp.float32)
    oh1 = (iota8 == i1).astype(jnp.float32)
    assign = oh0 + oh1
    r = lax.broadcasted_iota(jnp.int32, (TM, TM), 0)
    c = lax.broadcasted_iota(jnp.int32, (TM, TM), 1)
    tri = (r >= c).astype(jnp.float32)
    carry = jnp.zeros((1, E), jnp.float32)
    parts = []
    for b in range(T // TM):
        cum = jnp.dot(tri, assign[b * TM:(b + 1) * TM, :],
                      preferred_element_type=jnp.float32) + carry
        parts.append(cum)
        carry = cum[TM - 1:TM, :]
    incl = jnp.concatenate(parts, axis=0)
    cnt = carry
    tiles = jnp.ceil(cnt / TM)
    ue = (lax.broadcasted_iota(jnp.int32, (E, E), 0)
          <= lax.broadcasted_iota(jnp.int32, (E, E), 1)).astype(jnp.float32)
    cumt = jnp.dot(tiles, ue, preferred_element_type=jnp.float32)
    start_slot = (cumt - tiles) * TM
    pos = start_slot + incl - 1.0
    inv0 = jnp.sum(oh0 * pos, axis=1, keepdims=True)
    inv1 = jnp.sum(oh1 * pos, axis=1, keepdims=True)
    inv_ref[...] = jnp.concatenate([inv0, inv1], axis=1).astype(jnp.int32)
    w01_ref[...] = jnp.concatenate([w0, 1.0 - w0], axis=1)
    st_ref[...] = (cumt - tiles).astype(jnp.int32)
    sc_ref[...] = tiles.astype(jnp.int32)


_plan = pl.pallas_call(
    _plan_body,
    out_shape=[
        jax.ShapeDtypeStruct((T, 2), jnp.int32),
        jax.ShapeDtypeStruct((T, 2), jnp.float32),
        jax.ShapeDtypeStruct((1, E), jnp.int32),
        jax.ShapeDtypeStruct((1, E), jnp.int32),
    ],
)


@functools.cache
def _sc_mesh():
    return plsc.VectorSubcoreMesh(
        core_axis_name="c", subcore_axis_name="s",
        num_cores=NC, num_subcores=NS)


TPW = T // NW


@functools.cache
def _dispatch_kernel():
    @functools.partial(
        pl.kernel,
        out_type=jax.ShapeDtypeStruct((P, H), jnp.float32),
        mesh=_sc_mesh(),
        scratch_types=[
            pltpu.VMEM((K, TPW), jnp.int32),
            pltpu.VMEM((TPW, H), jnp.float32),
            pltpu.SemaphoreType.DMA,
        ],
        compiler_params=pltpu.CompilerParams(needs_layout_passes=False),
    )
    def dispatch(x_hbm, idx3_hbm, xs_hbm, idxw_v, rows_v, sem):
        c = lax.axis_index("c")
        s = lax.axis_index("s")
        wid = c * NS + s
        pltpu.sync_copy(x_hbm.at[pl.ds(wid * TPW, TPW)], rows_v)
        pltpu.sync_copy(idx3_hbm.at[wid], idxw_v)
        cps = [pltpu.async_copy(rows_v, xs_hbm.at[idxw_v.at[k]], sem)
               for k in range(K)]
        for cp in cps:
            cp.wait()

    return dispatch


def _ffn_body(st_ref, sc_ref, xs_hbm, w1a_ref, w1b_ref, w2a_ref, w2b_ref,
              ys_hbm, xbuf, ybuf, insem, outsem):
    e = pl.program_id(0)
    base = st_ref[e]
    n = sc_ref[e]

    def in_cp(i, slot):
        return pltpu.make_async_copy(
            xs_hbm.at[pl.ds((base + i) * TM, TM)], xbuf.at[slot],
            insem.at[slot])

    def out_cp(i, slot):
        return pltpu.make_async_copy(
            ybuf.at[slot], ys_hbm.at[pl.ds((base + i) * TM, TM)],
            outsem.at[slot])

    @pl.when(n > 0)
    def _():
        in_cp(0, 0).start()

    def loop_body(i, carry):
        slot = lax.rem(i, 2)
        nslot = lax.rem(i + 1, 2)

        @pl.when(i + 1 < n)
        def _():
            in_cp(i + 1, nslot).start()

        in_cp(i, slot).wait()
        xb = xbuf[slot]
        h = (jnp.dot(xb[:, :H // 2], w1a_ref[0, 0],
                     preferred_element_type=jnp.float32)
             + jnp.dot(xb[:, H // 2:], w1b_ref[0, 0],
                       preferred_element_type=jnp.float32))
        a = h[:, :FFN]
        b = h[:, FFN:]
        act = (a * lax.logistic(a)) * b
        y = (jnp.dot(act[:, :FFN // 2], w2a_ref[0, 0],
                     preferred_element_type=jnp.float32)
             + jnp.dot(act[:, FFN // 2:], w2b_ref[0, 0],
                       preferred_element_type=jnp.float32))

        @pl.when(i >= 2)
        def _():
            out_cp(i - 2, slot).wait()

        ybuf[slot] = y.astype(jnp.bfloat16)
        out_cp(i, slot).start()
        return carry

    lax.fori_loop(0, n, loop_body, 0)

    @pl.when(n >= 2)
    def _():
        out_cp(n - 2, lax.rem(n, 2)).wait()

    @pl.when(n >= 1)
    def _():
        out_cp(n - 1, lax.rem(n + 1, 2)).wait()


_ffn = pl.pallas_call(
    _ffn_body,
    grid_spec=pltpu.PrefetchScalarGridSpec(
        num_scalar_prefetch=2,
        grid=(E,),
        in_specs=[
            pl.BlockSpec(memory_space=pltpu.MemorySpace.HBM),
            pl.BlockSpec((1, 1, H // 2, F2), lambda e, st, sc: (e, 0, 0, 0)),
            pl.BlockSpec((1, 1, H // 2, F2), lambda e, st, sc: (e, 1, 0, 0)),
            pl.BlockSpec((1, 1, FFN // 2, H), lambda e, st, sc: (e, 0, 0, 0)),
            pl.BlockSpec((1, 1, FFN // 2, H), lambda e, st, sc: (e, 1, 0, 0)),
        ],
        out_specs=pl.BlockSpec(memory_space=pltpu.MemorySpace.HBM),
        scratch_shapes=[
            pltpu.VMEM((2, TM, H), jnp.float32),
            pltpu.VMEM((2, TM, H), jnp.bfloat16),
            pltpu.SemaphoreType.DMA((2,)),
            pltpu.SemaphoreType.DMA((2,)),
        ],
    ),
    out_shape=jax.ShapeDtypeStruct((P, H), jnp.bfloat16),
    compiler_params=pltpu.CompilerParams(
        dimension_semantics=("arbitrary",),
        vmem_limit_bytes=110 * 1024 * 1024),
)


@functools.cache
def _combine_kernel():
    @functools.partial(
        pl.kernel,
        out_type=jax.ShapeDtypeStruct((NPAIR, H // 2), jnp.int32),
        mesh=_sc_mesh(),
        scratch_types=[
            pltpu.VMEM((CPW,), jnp.int32),
            pltpu.VMEM((CPW, H // 2), jnp.int32),
            pltpu.SemaphoreType.DMA,
        ],
    )
    def combine(ys_hbm, slots_hbm, g_hbm, idx_v, rows_v, sem):
        c = lax.axis_index("c")
        s = lax.axis_index("s")
        base = (s * NC + c) * CPW
        pltpu.sync_copy(slots_hbm.at[pl.ds(base, CPW)], idx_v)
        pltpu.async_copy(ys_hbm.at[idx_v], rows_v, sem).wait()
        pltpu.sync_copy(rows_v, g_hbm.at[pl.ds(base, CPW)])

    return combine


def _mix_body(g_ref, gg_ref, w_ref, o_ref):
    w = w_ref[...]
    o_ref[...] = (g_ref[...].astype(jnp.float32) * w[:, 0:1]
                  + gg_ref[...].astype(jnp.float32) * w[:, 1:2])


_MIX_TB = 256
_mix = pl.pallas_call(
    _mix_body,
    grid=(T // _MIX_TB,),
    in_specs=[
        pl.BlockSpec((_MIX_TB, H), lambda i: (i, 0)),
        pl.BlockSpec((_MIX_TB, H), lambda i: (i + T // _MIX_TB, 0)),
        pl.BlockSpec((_MIX_TB, 2), lambda i: (i, 0)),
    ],
    out_specs=pl.BlockSpec((_MIX_TB, H), lambda i: (i, 0)),
    out_shape=jax.ShapeDtypeStruct((T, H), jnp.float32),
)


def kernel(hidden_states, W_router, w1, w2):
    Bv, Sv, Hv = hidden_states.shape
    x = hidden_states.reshape(Bv * Sv, Hv)
    inv, w01, st, sc = _plan(x, W_router)
    slots = jnp.concatenate([inv[:, 0], inv[:, 1]])
    idx3 = inv.reshape(NW, TPW, K).transpose(0, 2, 1)
    xs = _dispatch_kernel()(x, idx3)
    w1r = w1.reshape(E, 2, H // 2, F2)
    w2r = w2.reshape(E, 2, FFN // 2, H)
    ys = _ffn(st.reshape(E), sc.reshape(E), xs, w1r, w1r, w2r, w2r)
    ysi = lax.bitcast_convert_type(ys.reshape(P, H // 2, 2), jnp.int32)
    g = _combine_kernel()(ysi, slots)
    gb = lax.bitcast_convert_type(g, jnp.bfloat16).reshape(NPAIR, H)
    out = _mix(gb, gb, w01)
    return out.reshape(Bv, Sv, Hv)


# speedup vs baseline: 1.9389x; 1.9389x over previous
import functools

import jax
import jax.numpy as jnp
from jax import lax
from jax.experimental import pallas as pl
from jax.experimental.pallas import tpu as pltpu
from jax.experimental.pallas import tpu_sc as plsc

E = 8
H = 1024
FFN = 2048
F2 = 2 * FFN
T = 2048
K = 2
NPAIR = K * T

TM = 128
NT = 40
P = NT * TM

NC = 2
NS = 16
NW = NC * NS
HALF = P // NC
SLOTS_W = HALF // NS
GCH = 80
CPW = NPAIR // NW
CCH = 64


def _plan_body(x_ref, wr_ref, inv_ref, w01_ref, st_ref, sc_ref):
    x = x_ref[...]
    wr = wr_ref[...]
    mix = jnp.dot(x, wr, preferred_element_type=jnp.float32)
    q, k, v = mix[:, 0:E], mix[:, E:2 * E], mix[:, 2 * E:3 * E]
    cols = []
    for i in range(E):
        a = q[:, i:i + 1] * k
        m = jnp.max(a, axis=1, keepdims=True)
        ex = jnp.exp(a - m)
        cols.append(jnp.sum(ex * v, axis=1, keepdims=True)
                    / jnp.sum(ex, axis=1, keepdims=True))
    logits = jnp.concatenate(cols, axis=1)
    iota8 = lax.broadcasted_iota(jnp.int32, (T, E), 1)
    l0 = jnp.max(logits, axis=1, keepdims=True)
    i0 = jnp.min(jnp.where(logits == l0, iota8, E), axis=1, keepdims=True)
    rest = jnp.where(iota8 == i0, -jnp.inf, logits)
    l1 = jnp.max(rest, axis=1, keepdims=True)
    i1 = jnp.min(jnp.where(rest == l1, iota8, E), axis=1, keepdims=True)
    w0 = 1.0 / (1.0 + jnp.exp(l1 - l0))

    oh0 = (iota8 == i0).astype(jnp.float32)
    oh1 = (iota8 == i1).astype(jnp.float32)
    assign = oh0 + oh1
    r = lax.broadcasted_iota(jnp.int32, (TM, TM), 0)
    c = lax.broadcasted_iota(jnp.int32, (TM, TM), 1)
    tri = (r >= c).astype(jnp.float32)
    carry = jnp.zeros((1, E), jnp.float32)
    parts = []
    for b in range(T // TM):
        cum = jnp.dot(tri, assign[b * TM:(b + 1) * TM, :],
                      preferred_element_type=jnp.float32) + carry
        parts.append(cum)
        carry = cum[TM - 1:TM, :]
    incl = jnp.concatenate(parts, axis=0)
    cnt = carry
    tiles = jnp.ceil(cnt / TM)
    ue = (lax.broadcasted_iota(jnp.int32, (E, E), 0)
          <= lax.broadcasted_iota(jnp.int32, (E, E), 1)).astype(jnp.float32)
    cumt = jnp.dot(tiles, ue, preferred_element_type=jnp.float32)
    start_slot = (cumt - tiles) * TM
    pos = start_slot + incl - 1.0
    inv0 = jnp.sum(oh0 * pos, axis=1, keepdims=True)
    inv1 = jnp.sum(oh1 * pos, axis=1, keepdims=True)
    inv_ref[...] = jnp.concatenate([inv0, inv1], axis=1).astype(jnp.int32)
    w01_ref[...] = jnp.concatenate([w0, 1.0 - w0], axis=1)
    st_ref[...] = (cumt - tiles).astype(jnp.int32)
    sc_ref[...] = tiles.astype(jnp.int32)


_plan = pl.pallas_call(
    _plan_body,
    out_shape=[
        jax.ShapeDtypeStruct((T, 2), jnp.int32),
        jax.ShapeDtypeStruct((T, 2), jnp.float32),
        jax.ShapeDtypeStruct((1, E), jnp.int32),
        jax.ShapeDtypeStruct((1, E), jnp.int32),
    ],
)


@functools.cache
def _sc_mesh():
    return plsc.VectorSubcoreMesh(
        core_axis_name="c", subcore_axis_name="s",
        num_cores=NC, num_subcores=NS)


TPW = T // NW


@functools.cache
def _dispatch_kernel():
    @functools.partial(
        pl.kernel,
        out_type=jax.ShapeDtypeStruct((P, H), jnp.float32),
        mesh=_sc_mesh(),
        scratch_types=[
            pltpu.VMEM((K, TPW), jnp.int32),
            pltpu.VMEM((TPW, H), jnp.float32),
            pltpu.SemaphoreType.DMA,
        ],
        compiler_params=pltpu.CompilerParams(needs_layout_passes=False),
    )
    def dispatch(x_hbm, idx3_hbm, xs_hbm, idxw_v, rows_v, sem):
        c = lax.axis_index("c")
        s = lax.axis_index("s")
        wid = c * NS + s
        pltpu.sync_copy(x_hbm.at[pl.ds(wid * TPW, TPW)], rows_v)
        pltpu.sync_copy(idx3_hbm.at[wid], idxw_v)
        cps = [pltpu.async_copy(rows_v, xs_hbm.at[idxw_v.at[k]], sem)
               for k in range(K)]
        for cp in cps:
            cp.wait()

    return dispatch


def _ffn_body(st_ref, sc_ref, xs_hbm, w1a_ref, w1b_ref, w2a_ref, w2b_ref,
              ys_hbm, xbuf, ybuf, insem, outsem):
    e = pl.program_id(0)
    base = st_ref[e]
    n = sc_ref[e]

    def in_cp(i, slot):
        return pltpu.make_async_copy(
            xs_hbm.at[pl.ds((base + i) * TM, TM)], xbuf.at[slot],
            insem.at[slot])

    def out_cp(i, slot):
        return pltpu.make_async_copy(
            ybuf.at[slot], ys_hbm.at[pl.ds((base + i) * TM, TM)],
            outsem.at[slot])

    @pl.when(n > 0)
    def _():
        in_cp(0, 0).start()

    def loop_body(i, carry):
        slot = lax.rem(i, 2)
        nslot = lax.rem(i + 1, 2)

        @pl.when(i + 1 < n)
        def _():
            in_cp(i + 1, nslot).start()

        in_cp(i, slot).wait()
        xb = xbuf[slot]
        h = (jnp.dot(xb[:, :H // 2], w1a_ref[0, 0],
                     preferred_element_type=jnp.float32)
             + jnp.dot(xb[:, H // 2:], w1b_ref[0, 0],
                       preferred_element_type=jnp.float32))
        a = h[:, :FFN]
        b = h[:, FFN:]
        act = (a * lax.logistic(a)) * b
        y = (jnp.dot(act[:, :FFN // 2], w2a_ref[0, 0],
                     preferred_element_type=jnp.float32)
             + jnp.dot(act[:, FFN // 2:], w2b_ref[0, 0],
                       preferred_element_type=jnp.float32))

        @pl.when(i >= 2)
        def _():
            out_cp(i - 2, slot).wait()

        yb = y.astype(jnp.bfloat16)
        lo = lax.bitcast_convert_type(yb[:, :H // 2],
                                      jnp.uint16).astype(jnp.uint32)
        hi = lax.bitcast_convert_type(yb[:, H // 2:],
                                      jnp.uint16).astype(jnp.uint32)
        ybuf[slot] = lax.bitcast_convert_type(lo | (hi << 16), jnp.int32)
        out_cp(i, slot).start()
        return carry

    lax.fori_loop(0, n, loop_body, 0)

    @pl.when(n >= 2)
    def _():
        out_cp(n - 2, lax.rem(n, 2)).wait()

    @pl.when(n >= 1)
    def _():
        out_cp(n - 1, lax.rem(n + 1, 2)).wait()


_ffn = pl.pallas_call(
    _ffn_body,
    grid_spec=pltpu.PrefetchScalarGridSpec(
        num_scalar_prefetch=2,
        grid=(E,),
        in_specs=[
            pl.BlockSpec(memory_space=pltpu.MemorySpace.HBM),
            pl.BlockSpec((1, 1, H // 2, F2), lambda e, st, sc: (e, 0, 0, 0)),
            pl.BlockSpec((1, 1, H // 2, F2), lambda e, st, sc: (e, 1, 0, 0)),
            pl.BlockSpec((1, 1, FFN // 2, H), lambda e, st, sc: (e, 0, 0, 0)),
            pl.BlockSpec((1, 1, FFN // 2, H), lambda e, st, sc: (e, 1, 0, 0)),
        ],
        out_specs=pl.BlockSpec(memory_space=pltpu.MemorySpace.HBM),
        scratch_shapes=[
            pltpu.VMEM((2, TM, H), jnp.float32),
            pltpu.VMEM((2, TM, H // 2), jnp.int32),
            pltpu.SemaphoreType.DMA((2,)),
            pltpu.SemaphoreType.DMA((2,)),
        ],
    ),
    out_shape=jax.ShapeDtypeStruct((P, H // 2), jnp.int32),
    compiler_params=pltpu.CompilerParams(
        dimension_semantics=("arbitrary",),
        vmem_limit_bytes=110 * 1024 * 1024),
)


@functools.cache
def _combine_kernel():
    @functools.partial(
        pl.kernel,
        out_type=jax.ShapeDtypeStruct((NPAIR, H // 2), jnp.int32),
        mesh=_sc_mesh(),
        scratch_types=[
            pltpu.VMEM((CPW,), jnp.int32),
            pltpu.VMEM((CPW, H // 2), jnp.int32),
            pltpu.SemaphoreType.DMA,
        ],
    )
    def combine(ys_hbm, slots_hbm, g_hbm, idx_v, rows_v, sem):
        c = lax.axis_index("c")
        s = lax.axis_index("s")
        base = (s * NC + c) * CPW
        pltpu.sync_copy(slots_hbm.at[pl.ds(base, CPW)], idx_v)
        pltpu.async_copy(ys_hbm.at[idx_v], rows_v, sem).wait()
        pltpu.sync_copy(rows_v, g_hbm.at[pl.ds(base, CPW)])

    return combine


def _unpack_bf16_pair(gi32):
    u = lax.bitcast_convert_type(gi32, jnp.uint32)
    lo = lax.bitcast_convert_type((u & 0xFFFF).astype(jnp.uint16),
                                  jnp.bfloat16).astype(jnp.float32)
    hi = lax.bitcast_convert_type((u >> 16).astype(jnp.uint16),
                                  jnp.bfloat16).astype(jnp.float32)
    return jnp.concatenate([lo, hi], axis=1)


def _mix_body(g_ref, gg_ref, w_ref, o_ref):
    w = w_ref[...]
    g0 = _unpack_bf16_pair(g_ref[...])
    g1 = _unpack_bf16_pair(gg_ref[...])
    o_ref[...] = g0 * w[:, 0:1] + g1 * w[:, 1:2]


_MIX_TB = 256
_mix = pl.pallas_call(
    _mix_body,
    grid=(T // _MIX_TB,),
    in_specs=[
        pl.BlockSpec((_MIX_TB, H // 2), lambda i: (i, 0)),
        pl.BlockSpec((_MIX_TB, H // 2), lambda i: (i + T // _MIX_TB, 0)),
        pl.BlockSpec((_MIX_TB, 2), lambda i: (i, 0)),
    ],
    out_specs=pl.BlockSpec((_MIX_TB, H), lambda i: (i, 0)),
    out_shape=jax.ShapeDtypeStruct((T, H), jnp.float32),
)


def kernel(hidden_states, W_router, w1, w2):
    Bv, Sv, Hv = hidden_states.shape
    x = hidden_states.reshape(Bv * Sv, Hv)
    inv, w01, st, sc = _plan(x, W_router)
    slots = jnp.concatenate([inv[:, 0], inv[:, 1]])
    idx3 = inv.reshape(NW, TPW, K).transpose(0, 2, 1)
    xs = _dispatch_kernel()(x, idx3)
    w1r = w1.reshape(E, 2, H // 2, F2)
    w2r = w2.reshape(E, 2, FFN // 2, H)
    ys = _ffn(st.reshape(E), sc.reshape(E), xs, w1r, w1r, w2r, w2r)
    g = _combine_kernel()(ys, slots)
    out = _mix(g, g, w01)
    return out.reshape(Bv, Sv, Hv)


# speedup vs baseline: 2.0903x; 1.0781x over previous
import functools

import jax
import jax.numpy as jnp
from jax import lax
from jax.experimental import pallas as pl
from jax.experimental.pallas import tpu as pltpu
from jax.experimental.pallas import tpu_sc as plsc

E = 8
H = 1024
FFN = 2048
F2 = 2 * FFN
T = 2048
K = 2
NPAIR = K * T

TM = 128
NT = 40
P = NT * TM

NC = 2
NS = 16
NW = NC * NS
HALF = P // NC
SLOTS_W = HALF // NS
GCH = 80
CPW = NPAIR // NW
CCH = 64


def _plan_body(x_ref, wr_ref, inv_ref, w01_ref, te_ref, tv_ref):
    x = x_ref[...]
    wr = wr_ref[...]
    mix = jnp.dot(x, wr, preferred_element_type=jnp.float32)
    q, k, v = mix[:, 0:E], mix[:, E:2 * E], mix[:, 2 * E:3 * E]
    cols = []
    for i in range(E):
        a = q[:, i:i + 1] * k
        m = jnp.max(a, axis=1, keepdims=True)
        ex = jnp.exp(a - m)
        cols.append(jnp.sum(ex * v, axis=1, keepdims=True)
                    / jnp.sum(ex, axis=1, keepdims=True))
    logits = jnp.concatenate(cols, axis=1)
    iota8 = lax.broadcasted_iota(jnp.int32, (T, E), 1)
    l0 = jnp.max(logits, axis=1, keepdims=True)
    i0 = jnp.min(jnp.where(logits == l0, iota8, E), axis=1, keepdims=True)
    rest = jnp.where(iota8 == i0, -jnp.inf, logits)
    l1 = jnp.max(rest, axis=1, keepdims=True)
    i1 = jnp.min(jnp.where(rest == l1, iota8, E), axis=1, keepdims=True)
    w0 = 1.0 / (1.0 + jnp.exp(l1 - l0))

    oh0 = (iota8 == i0).astype(jnp.float32)
    oh1 = (iota8 == i1).astype(jnp.float32)
    assign = oh0 + oh1
    r = lax.broadcasted_iota(jnp.int32, (TM, TM), 0)
    c = lax.broadcasted_iota(jnp.int32, (TM, TM), 1)
    tri = (r >= c).astype(jnp.float32)
    carry = jnp.zeros((1, E), jnp.float32)
    parts = []
    for b in range(T // TM):
        cum = jnp.dot(tri, assign[b * TM:(b + 1) * TM, :],
                      preferred_element_type=jnp.float32) + carry
        parts.append(cum)
        carry = cum[TM - 1:TM, :]
    incl = jnp.concatenate(parts, axis=0)
    cnt = carry
    tiles = jnp.ceil(cnt / TM)
    ue = (lax.broadcasted_iota(jnp.int32, (E, E), 0)
          <= lax.broadcasted_iota(jnp.int32, (E, E), 1)).astype(jnp.float32)
    cumt = jnp.dot(tiles, ue, preferred_element_type=jnp.float32)
    start_slot = (cumt - tiles) * TM
    pos = start_slot + incl - 1.0
    inv0 = jnp.sum(oh0 * pos, axis=1, keepdims=True)
    inv1 = jnp.sum(oh1 * pos, axis=1, keepdims=True)
    inv_ref[...] = jnp.concatenate([inv0, inv1], axis=1).astype(jnp.int32)
    w01_ref[...] = jnp.concatenate([w0, 1.0 - w0], axis=1)
    jt = lax.broadcasted_iota(jnp.int32, (NT, E), 0).astype(jnp.float32)
    raw = jnp.sum((cumt <= jt).astype(jnp.float32), axis=1, keepdims=True)
    last_active = jnp.sum((cumt < cumt[0:1, E - 1:E]).astype(jnp.float32),
                          axis=1, keepdims=True)
    te_ref[...] = jnp.minimum(raw, last_active).astype(jnp.int32)
    tv_ref[...] = (jt[:, 0:1] < cumt[0:1, E - 1:E]).astype(jnp.int32)


_plan = pl.pallas_call(
    _plan_body,
    out_shape=[
        jax.ShapeDtypeStruct((T, 2), jnp.int32),
        jax.ShapeDtypeStruct((T, 2), jnp.float32),
        jax.ShapeDtypeStruct((NT, 1), jnp.int32),
        jax.ShapeDtypeStruct((NT, 1), jnp.int32),
    ],
)


@functools.cache
def _sc_mesh():
    return plsc.VectorSubcoreMesh(
        core_axis_name="c", subcore_axis_name="s",
        num_cores=NC, num_subcores=NS)


TPW = T // NW


@functools.cache
def _dispatch_kernel():
    @functools.partial(
        pl.kernel,
        out_type=jax.ShapeDtypeStruct((P, H), jnp.float32),
        mesh=_sc_mesh(),
        scratch_types=[
            pltpu.VMEM((K, TPW), jnp.int32),
            pltpu.VMEM((TPW, H), jnp.float32),
            pltpu.SemaphoreType.DMA,
        ],
        compiler_params=pltpu.CompilerParams(needs_layout_passes=False),
    )
    def dispatch(x_hbm, idx3_hbm, xs_hbm, idxw_v, rows_v, sem):
        c = lax.axis_index("c")
        s = lax.axis_index("s")
        wid = c * NS + s
        pltpu.sync_copy(x_hbm.at[pl.ds(wid * TPW, TPW)], rows_v)
        pltpu.sync_copy(idx3_hbm.at[wid], idxw_v)
        cps = [pltpu.async_copy(rows_v, xs_hbm.at[idxw_v.at[k]], sem)
               for k in range(K)]
        for cp in cps:
            cp.wait()

    return dispatch


def _ffn_body(te_ref, tv_ref, xs_ref, w1_ref, w2_ref, ys_ref):
    j = pl.program_id(0)

    @pl.when(tv_ref[j] == 1)
    def _():
        xb = xs_ref[...]
        h = jnp.dot(xb, w1_ref[0], preferred_element_type=jnp.float32)
        a = h[:, :FFN]
        b = h[:, FFN:]
        act = (a * lax.logistic(a)) * b
        y = jnp.dot(act, w2_ref[0], preferred_element_type=jnp.float32)
        yb = y.astype(jnp.bfloat16)
        lo = lax.bitcast_convert_type(yb[:, :H // 2],
                                      jnp.uint16).astype(jnp.uint32)
        hi = lax.bitcast_convert_type(yb[:, H // 2:],
                                      jnp.uint16).astype(jnp.uint32)
        ys_ref[...] = lax.bitcast_convert_type(lo | (hi << 16), jnp.int32)


_ffn = pl.pallas_call(
    _ffn_body,
    grid_spec=pltpu.PrefetchScalarGridSpec(
        num_scalar_prefetch=2,
        grid=(NT,),
        in_specs=[
            pl.BlockSpec((TM, H), lambda j, te, tv: (j, 0)),
            pl.BlockSpec((1, H, F2), lambda j, te, tv: (te[j], 0, 0)),
            pl.BlockSpec((1, FFN, H), lambda j, te, tv: (te[j], 0, 0)),
        ],
        out_specs=pl.BlockSpec((TM, H // 2), lambda j, te, tv: (j, 0)),
    ),
    out_shape=jax.ShapeDtypeStruct((P, H // 2), jnp.int32),
    compiler_params=pltpu.CompilerParams(
        dimension_semantics=("arbitrary",)),
)


@functools.cache
def _combine_kernel():
    @functools.partial(
        pl.kernel,
        out_type=jax.ShapeDtypeStruct((NPAIR, H // 2), jnp.int32),
        mesh=_sc_mesh(),
        scratch_types=[
            pltpu.VMEM((CPW,), jnp.int32),
            pltpu.VMEM((CPW, H // 2), jnp.int32),
            pltpu.SemaphoreType.DMA,
        ],
    )
    def combine(ys_hbm, slots_hbm, g_hbm, idx_v, rows_v, sem):
        c = lax.axis_index("c")
        s = lax.axis_index("s")
        base = (s * NC + c) * CPW
        pltpu.sync_copy(slots_hbm.at[pl.ds(base, CPW)], idx_v)
        pltpu.async_copy(ys_hbm.at[idx_v], rows_v, sem).wait()
        pltpu.sync_copy(rows_v, g_hbm.at[pl.ds(base, CPW)])

    return combine


def _unpack_bf16_pair(gi32):
    u = lax.bitcast_convert_type(gi32, jnp.uint32)
    lo = lax.bitcast_convert_type((u & 0xFFFF).astype(jnp.uint16),
                                  jnp.bfloat16).astype(jnp.float32)
    hi = lax.bitcast_convert_type((u >> 16).astype(jnp.uint16),
                                  jnp.bfloat16).astype(jnp.float32)
    return jnp.concatenate([lo, hi], axis=1)


def _mix_body(g_ref, gg_ref, w_ref, o_ref):
    w = w_ref[...]
    g0 = _unpack_bf16_pair(g_ref[...])
    g1 = _unpack_bf16_pair(gg_ref[...])
    o_ref[...] = g0 * w[:, 0:1] + g1 * w[:, 1:2]


_MIX_TB = 256
_mix = pl.pallas_call(
    _mix_body,
    grid=(T // _MIX_TB,),
    in_specs=[
        pl.BlockSpec((_MIX_TB, H // 2), lambda i: (i, 0)),
        pl.BlockSpec((_MIX_TB, H // 2), lambda i: (i + T // _MIX_TB, 0)),
        pl.BlockSpec((_MIX_TB, 2), lambda i: (i, 0)),
    ],
    out_specs=pl.BlockSpec((_MIX_TB, H), lambda i: (i, 0)),
    out_shape=jax.ShapeDtypeStruct((T, H), jnp.float32),
)


def kernel(hidden_states, W_router, w1, w2):
    Bv, Sv, Hv = hidden_states.shape
    x = hidden_states.reshape(Bv * Sv, Hv)
    inv, w01, te, tv = _plan(x, W_router)
    slots = jnp.concatenate([inv[:, 0], inv[:, 1]])
    idx3 = inv.reshape(NW, TPW, K).transpose(0, 2, 1)
    xs = _dispatch_kernel()(x, idx3)
    ys = _ffn(te.reshape(NT), tv.reshape(NT), xs, w1, w2)
    g = _combine_kernel()(ys, slots)
    out = _mix(g, g, w01)
    return out.reshape(Bv, Sv, Hv)


# speedup vs baseline: 2.1487x; 1.0279x over previous
import functools

import jax
import jax.numpy as jnp
from jax import lax
from jax.experimental import pallas as pl
from jax.experimental.pallas import tpu as pltpu
from jax.experimental.pallas import tpu_sc as plsc

E = 8
H = 1024
FFN = 2048
F2 = 2 * FFN
T = 2048
K = 2
NPAIR = K * T

TM = 128
NT = 40
P = NT * TM

NC = 2
NS = 16
NW = NC * NS
HALF = P // NC
SLOTS_W = HALF // NS
GCH = 80
CPW = NPAIR // NW
CCH = 64


def _plan_body(x_ref, wr_ref, inv_ref, w01_ref, te_ref, tv_ref):
    x = x_ref[...]
    wr = wr_ref[...]
    mix = jnp.dot(x, wr, preferred_element_type=jnp.float32)
    q, k, v = mix[:, 0:E], mix[:, E:2 * E], mix[:, 2 * E:3 * E]
    rr = lax.broadcasted_iota(jnp.int32, (E, E * E), 0)
    cc = lax.broadcasted_iota(jnp.int32, (E, E * E), 1)
    rq = (rr == cc // E).astype(jnp.float32)
    rk = (rr == cc % E).astype(jnp.float32)
    scores = (jnp.dot(q, rq, preferred_element_type=jnp.float32)
              * jnp.dot(k, rk, preferred_element_type=jnp.float32))
    m = jnp.max(scores, axis=1, keepdims=True)
    ex = jnp.exp(scores - m)
    vrep = jnp.dot(v, rk, preferred_element_type=jnp.float32)
    g1 = lax.broadcasted_iota(jnp.int32, (E * E, E), 0)
    g2 = lax.broadcasted_iota(jnp.int32, (E * E, E), 1)
    grp = (g1 // E == g2).astype(jnp.float32)
    num = jnp.dot(ex * vrep, grp, preferred_element_type=jnp.float32)
    den = jnp.dot(ex, grp, preferred_element_type=jnp.float32)
    logits = num / den
    iota8 = lax.broadcasted_iota(jnp.int32, (T, E), 1)
    l0 = jnp.max(logits, axis=1, keepdims=True)
    i0 = jnp.min(jnp.where(logits == l0, iota8, E), axis=1, keepdims=True)
    rest = jnp.where(iota8 == i0, -jnp.inf, logits)
    l1 = jnp.max(rest, axis=1, keepdims=True)
    i1 = jnp.min(jnp.where(rest == l1, iota8, E), axis=1, keepdims=True)
    w0 = 1.0 / (1.0 + jnp.exp(l1 - l0))

    oh0 = (iota8 == i0).astype(jnp.float32)
    oh1 = (iota8 == i1).astype(jnp.float32)
    assign = oh0 + oh1
    r = lax.broadcasted_iota(jnp.int32, (TM, TM), 0)
    c = lax.broadcasted_iota(jnp.int32, (TM, TM), 1)
    tri = (r >= c).astype(jnp.float32)
    carry = jnp.zeros((1, E), jnp.float32)
    parts = []
    for b in range(T // TM):
        cum = jnp.dot(tri, assign[b * TM:(b + 1) * TM, :],
                      preferred_element_type=jnp.float32) + carry
        parts.append(cum)
        carry = cum[TM - 1:TM, :]
    incl = jnp.concatenate(parts, axis=0)
    cnt = carry
    tiles = jnp.ceil(cnt / TM)
    ue = (lax.broadcasted_iota(jnp.int32, (E, E), 0)
          <= lax.broadcasted_iota(jnp.int32, (E, E), 1)).astype(jnp.float32)
    cumt = jnp.dot(tiles, ue, preferred_element_type=jnp.float32)
    start_slot = (cumt - tiles) * TM
    pos = start_slot + incl - 1.0
    inv0 = jnp.sum(oh0 * pos, axis=1, keepdims=True)
    inv1 = jnp.sum(oh1 * pos, axis=1, keepdims=True)
    inv_ref[...] = jnp.concatenate([inv0, inv1], axis=1).astype(jnp.int32)
    w01_ref[...] = jnp.concatenate([w0, 1.0 - w0], axis=1)
    jt = lax.broadcasted_iota(jnp.int32, (NT, E), 0).astype(jnp.float32)
    raw = jnp.sum((cumt <= jt).astype(jnp.float32), axis=1, keepdims=True)
    last_active = jnp.sum((cumt < cumt[0:1, E - 1:E]).astype(jnp.float32),
                          axis=1, keepdims=True)
    te_ref[...] = jnp.minimum(raw, last_active).astype(jnp.int32)
    tv_ref[...] = (jt[:, 0:1] < cumt[0:1, E - 1:E]).astype(jnp.int32)


_plan = pl.pallas_call(
    _plan_body,
    out_shape=[
        jax.ShapeDtypeStruct((T, 2), jnp.int32),
        jax.ShapeDtypeStruct((T, 2), jnp.float32),
        jax.ShapeDtypeStruct((NT, 1), jnp.int32),
        jax.ShapeDtypeStruct((NT, 1), jnp.int32),
    ],
)


@functools.cache
def _sc_mesh():
    return plsc.VectorSubcoreMesh(
        core_axis_name="c", subcore_axis_name="s",
        num_cores=NC, num_subcores=NS)


TPW = T // NW


@functools.cache
def _dispatch_kernel():
    @functools.partial(
        pl.kernel,
        out_type=jax.ShapeDtypeStruct((P, H), jnp.float32),
        mesh=_sc_mesh(),
        scratch_types=[
            pltpu.VMEM((K, TPW), jnp.int32),
            pltpu.VMEM((TPW, H), jnp.float32),
            pltpu.SemaphoreType.DMA,
        ],
        compiler_params=pltpu.CompilerParams(needs_layout_passes=False),
    )
    def dispatch(x_hbm, idx3_hbm, xs_hbm, idxw_v, rows_v, sem):
        c = lax.axis_index("c")
        s = lax.axis_index("s")
        wid = c * NS + s
        pltpu.sync_copy(x_hbm.at[pl.ds(wid * TPW, TPW)], rows_v)
        pltpu.sync_copy(idx3_hbm.at[wid], idxw_v)
        cps = [pltpu.async_copy(rows_v, xs_hbm.at[idxw_v.at[k]], sem)
               for k in range(K)]
        for cp in cps:
            cp.wait()

    return dispatch


def _ffn_body(te_ref, tv_ref, xs_ref, w1_ref, w2_ref, ys_ref):
    j = pl.program_id(0)

    @pl.when(tv_ref[j] == 1)
    def _():
        xb = xs_ref[...]
        h = jnp.dot(xb, w1_ref[0], preferred_element_type=jnp.float32)
        a = h[:, :FFN]
        b = h[:, FFN:]
        act = (a * lax.logistic(a)) * b
        y = jnp.dot(act, w2_ref[0], preferred_element_type=jnp.float32)
        yb = y.astype(jnp.bfloat16)
        lo = lax.bitcast_convert_type(yb[:, :H // 2],
                                      jnp.uint16).astype(jnp.uint32)
        hi = lax.bitcast_convert_type(yb[:, H // 2:],
                                      jnp.uint16).astype(jnp.uint32)
        ys_ref[...] = lax.bitcast_convert_type(lo | (hi << 16), jnp.int32)


_ffn = pl.pallas_call(
    _ffn_body,
    grid_spec=pltpu.PrefetchScalarGridSpec(
        num_scalar_prefetch=2,
        grid=(NT,),
        in_specs=[
            pl.BlockSpec((TM, H), lambda j, te, tv: (j, 0)),
            pl.BlockSpec((1, H, F2), lambda j, te, tv: (te[j], 0, 0)),
            pl.BlockSpec((1, FFN, H), lambda j, te, tv: (te[j], 0, 0)),
        ],
        out_specs=pl.BlockSpec((TM, H // 2), lambda j, te, tv: (j, 0)),
    ),
    out_shape=jax.ShapeDtypeStruct((P, H // 2), jnp.int32),
    compiler_params=pltpu.CompilerParams(
        dimension_semantics=("arbitrary",)),
)


@functools.cache
def _combine_kernel():
    @functools.partial(
        pl.kernel,
        out_type=jax.ShapeDtypeStruct((NPAIR, H // 2), jnp.int32),
        mesh=_sc_mesh(),
        scratch_types=[
            pltpu.VMEM((CPW,), jnp.int32),
            pltpu.VMEM((CPW, H // 2), jnp.int32),
            pltpu.SemaphoreType.DMA,
        ],
    )
    def combine(ys_hbm, slots_hbm, g_hbm, idx_v, rows_v, sem):
        c = lax.axis_index("c")
        s = lax.axis_index("s")
        base = (s * NC + c) * CPW
        pltpu.sync_copy(slots_hbm.at[pl.ds(base, CPW)], idx_v)
        pltpu.async_copy(ys_hbm.at[idx_v], rows_v, sem).wait()
        pltpu.sync_copy(rows_v, g_hbm.at[pl.ds(base, CPW)])

    return combine


def _unpack_bf16_pair(gi32):
    u = lax.bitcast_convert_type(gi32, jnp.uint32)
    lo = lax.bitcast_convert_type((u & 0xFFFF).astype(jnp.uint16),
                                  jnp.bfloat16).astype(jnp.float32)
    hi = lax.bitcast_convert_type((u >> 16).astype(jnp.uint16),
                                  jnp.bfloat16).astype(jnp.float32)
    return jnp.concatenate([lo, hi], axis=1)


def _mix_body(g_ref, w_ref, o_ref):
    w = w_ref[...]
    u = g_ref[...]
    g0 = _unpack_bf16_pair(u[:, :H // 2])
    g1 = _unpack_bf16_pair(u[:, H // 2:])
    o_ref[...] = g0 * w[:, 0:1] + g1 * w[:, 1:2]


_MIX_TB = 256
_mix = pl.pallas_call(
    _mix_body,
    grid=(T // _MIX_TB,),
    in_specs=[
        pl.BlockSpec((_MIX_TB, H), lambda i: (i, 0)),
        pl.BlockSpec((_MIX_TB, 2), lambda i: (i, 0)),
    ],
    out_specs=pl.BlockSpec((_MIX_TB, H), lambda i: (i, 0)),
    out_shape=jax.ShapeDtypeStruct((T, H), jnp.float32),
)


def kernel(hidden_states, W_router, w1, w2):
    Bv, Sv, Hv = hidden_states.shape
    x = hidden_states.reshape(Bv * Sv, Hv)
    inv, w01, te, tv = _plan(x, W_router)
    slots = inv.reshape(NPAIR)
    idx3 = inv.reshape(NW, TPW, K).transpose(0, 2, 1)
    xs = _dispatch_kernel()(x, idx3)
    ys = _ffn(te.reshape(NT), tv.reshape(NT), xs, w1, w2)
    g = _combine_kernel()(ys, slots)
    out = _mix(g.reshape(T, H), w01)
    return out.reshape(Bv, Sv, Hv)
